# baseline (speedup 1.0000x reference)
import functools

import jax
import jax.numpy as jnp
from jax import lax
from jax.experimental import pallas as pl
from jax.experimental.pallas import tpu as pltpu
from jax.experimental.pallas import tpu_sc as plsc

B, N = 4, 4096
C = 64
K = 16
G = 6
NCLS = 13
EPS = 1e-5
ROWS = B * N
RT = 2048
NR = ROWS // RT
T = 256
NT = N // T

_HI = jax.lax.Precision.HIGHEST


def _bdot(a, b):
    return jnp.dot(a.astype(jnp.bfloat16), b.astype(jnp.bfloat16),
                   preferred_element_type=jnp.float32)


def _full(shape):
    return pl.BlockSpec(shape, lambda *args: (0,) * len(shape))


def _rows(shape):
    return pl.BlockSpec(shape, lambda i: (i, 0))


def _k1(feat_ref, w_ref, b_ref, y1_ref, s_ref):
    y1 = _bdot(feat_ref[...], w_ref[...]) + b_ref[...]
    y1_ref[...] = y1

    @pl.when(pl.program_id(0) == 0)
    def _():
        s_ref[...] = jnp.zeros_like(s_ref)

    s_ref[0:1, :] += jnp.sum(y1, axis=0, keepdims=True)
    s_ref[1:2, :] += jnp.sum(y1 * y1, axis=0, keepdims=True)


def _k2(y1_ref, a1_ref, w_ref, b_ref, y2_ref, s_ref):
    h = jnp.maximum(y1_ref[...] * a1_ref[0:1, :] + a1_ref[1:2, :], 0.0)
    y2 = _bdot(h, w_ref[...]) + b_ref[...]
    y2_ref[...] = y2

    @pl.when(pl.program_id(0) == 0)
    def _():
        s_ref[...] = jnp.zeros_like(s_ref)

    s_ref[0:1, :] += jnp.sum(y2, axis=0, keepdims=True)
    s_ref[1:2, :] += jnp.sum(y2 * y2, axis=0, keepdims=True)


def _k3(y2_ref, feat_ref, a2_ref, scw_ref, shw_ref, qw_ref, kw_ref,
        bdw_ref, rp2_ref, geom_ref, q_ref, kf_ref, qw2_ref, y3_ref, s_ref):
    g0 = jnp.maximum(y2_ref[...] * a2_ref[0:1, :] + a2_ref[1:2, :], 0.0)
    f = feat_ref[...]
    scale = _bdot(f, scw_ref[...]) + a2_ref[2:3, :]
    shift = _bdot(f, shw_ref[...]) + a2_ref[3:4, :]
    geom = g0 * (scale + 1.0) + shift
    geom_ref[...] = geom
    q = _bdot(geom, qw_ref[...]) + a2_ref[4:5, :]
    q_ref[...] = q
    kf_ref[...] = _bdot(geom, kw_ref[...]) + a2_ref[5:6, :]
    qw2_ref[...] = jnp.dot(q, rp2_ref[...], precision=_HI)
    y3 = _bdot(geom, bdw_ref[...]) + a2_ref[6:7, 0:32]
    y3_ref[...] = y3

    @pl.when(pl.program_id(0) == 0)
    def _():
        s_ref[...] = jnp.zeros_like(s_ref)

    s_ref[0:1, :] += jnp.sum(y3, axis=0, keepdims=True)
    s_ref[1:2, :] += jnp.sum(y3 * y3, axis=0, keepdims=True)


def _k5(xt_ref, xall_ref, kidx_ref, kidxg_ref):
    xt = xt_ref[0]
    xa = xall_ref[0]
    sq_t = (xt[:, 0:1] * xt[:, 0:1] + xt[:, 1:2] * xt[:, 1:2]) \
        + xt[:, 2:3] * xt[:, 2:3]
    sq_a = (xa[0:1, :] * xa[0:1, :] + xa[1:2, :] * xa[1:2, :]) \
        + xa[2:3, :] * xa[2:3, :]
    e = _bdot(xt, xa)
    d2 = (sq_t + sq_a) - 2.0 * e

    lanef = jax.lax.broadcasted_iota(jnp.int32, (T, N), 1).astype(jnp.float32)
    lane16 = jax.lax.broadcasted_iota(jnp.int32, (T, K), 1)

    def body(k, carry):
        d2, kidxf = carry
        mv = jnp.min(d2, axis=1, keepdims=True)
        cand = jnp.where(d2 == mv, lanef, jnp.inf)
        idxk = jnp.min(cand, axis=1, keepdims=True)
        onehot = cand == idxk
        kidxf = jnp.where(lane16 == k, idxk, kidxf)
        d2 = jnp.where(onehot, jnp.inf, d2)
        return d2, kidxf

    _, kidxf = jax.lax.fori_loop(
        0, K, body, (d2, jnp.zeros((T, K), jnp.float32)))

    kidx = kidxf.astype(jnp.int32)
    kidx_ref[0] = kidx
    kidxg_ref[0] = kidx + pl.program_id(0) * N


PW = (B * N * K) // 32


def _sc_gather(x0_hbm, x1_hbm, x2_hbm, idx_hbm, o0_hbm, o1_hbm, o2_hbm,
               x0_v, x1_v, x2_v, idx_v, o0_v, o1_v, o2_v):
    wid = lax.axis_index("s") * 2 + lax.axis_index("c")
    base = wid * PW
    pltpu.sync_copy(x0_hbm, x0_v)
    pltpu.sync_copy(x1_hbm, x1_v)
    pltpu.sync_copy(x2_hbm, x2_v)
    pltpu.sync_copy(idx_hbm.at[pl.ds(base, PW)], idx_v)

    def body(j, _):
        for u in range(4):
            s = pl.ds(j * 64 + u * 16, 16)
            iv = idx_v[s]
            o0_v[s] = plsc.load_gather(x0_v, [iv])
            o1_v[s] = plsc.load_gather(x1_v, [iv])
            o2_v[s] = plsc.load_gather(x2_v, [iv])
        return 0

    lax.fori_loop(0, PW // 64, body, 0)
    pltpu.sync_copy(o0_v, o0_hbm.at[pl.ds(base, PW)])
    pltpu.sync_copy(o1_v, o1_hbm.at[pl.ds(base, PW)])
    pltpu.sync_copy(o2_v, o2_hbm.at[pl.ds(base, PW)])


CH = 16384


def _sc_gather2(gco_hbm, idx_hbm, val_hbm, t0_v, t1_v, idx_v, o0_v, o1_v):
    wid = lax.axis_index("s") * 2 + lax.axis_index("c")
    c0 = wid * 2
    pltpu.sync_copy(gco_hbm.at[c0], t0_v)
    pltpu.sync_copy(gco_hbm.at[c0 + 1], t1_v)
    for cc in range(16):
        base = cc * CH
        pltpu.sync_copy(idx_hbm.at[pl.ds(base, CH)], idx_v)

        def gbody(j, _):
            for u in range(4):
                s = pl.ds(j * 64 + u * 16, 16)
                iv = idx_v[s]
                o0_v[s] = plsc.load_gather(t0_v, [iv])
                o1_v[s] = plsc.load_gather(t1_v, [iv])
            return 0

        lax.fori_loop(0, CH // 64, gbody, 0)
        pltpu.sync_copy(o0_v, val_hbm.at[c0, pl.ds(base, CH)])
        pltpu.sync_copy(o1_v, val_hbm.at[c0 + 1, pl.ds(base, CH)])


def _k5b(xt_ref, x0_ref, x1_ref, x2_ref, xg_ref, st_ref):
    xt = xt_ref[0]
    xg0 = x0_ref[0]
    xg1 = x1_ref[0]
    xg2 = x2_ref[0]
    xg_ref[0, 0] = xg0
    xg_ref[0, 1] = xg1
    xg_ref[0, 2] = xg2

    rx = xt[:, 0:1] - xg0
    ry = xt[:, 1:2] - xg1
    rz = xt[:, 2:3] - xg2

    @pl.when((pl.program_id(0) == 0) & (pl.program_id(1) == 0))
    def _():
        st_ref[...] = jnp.zeros_like(st_ref)

    def acc(j, v):
        st_ref[j:j + 1, :] += jnp.broadcast_to(jnp.sum(v), (1, 128))

    acc(0, rx)
    acc(1, ry)
    acc(2, rz)
    acc(3, rx * rx)
    acc(4, ry * ry)
    acc(5, rz * rz)
    acc(6, rx * ry)
    acc(7, rx * rz)
    acc(8, ry * rz)


def _k6(xt_ref, xg_ref, qt_ref, qw2_ref, val_ref, kg_ref, y3_ref,
        pep_ref, bdp_ref, clw_ref, clb_ref,
        aff_ref, oft_ref, lgt_ref, bd_ref):
    xt = xt_ref[0]
    rx = xt[:, 0:1] - xg_ref[0, 0]
    ry = xt[:, 1:2] - xg_ref[0, 1]
    rz = xt[:, 2:3] - xg_ref[0, 2]
    a0 = pep_ref[0:1, :].reshape(1, 1, C)
    a1 = pep_ref[1:2, :].reshape(1, 1, C)
    a2 = pep_ref[2:3, :].reshape(1, 1, C)
    dv = pep_ref[3:4, :].reshape(1, 1, C)
    pe = rx[:, :, None] * a0 + ry[:, :, None] * a1
    pe = pe + rz[:, :, None] * a2 + dv
    pe = jnp.maximum(pe, 0.0)
    qw2 = qw2_ref[0]
    posq = jnp.sum(pe * qw2[:, None, :], axis=2)
    qc = qt_ref[0]
    aq = jnp.sum(kg_ref[:, 0] * qc[:, :, None], axis=0)
    attn = (aq + posq) * 0.125
    attn = attn - jnp.max(attn, axis=1, keepdims=True)
    ex = jnp.exp(attn)
    aff = ex / jnp.sum(ex, axis=1, keepdims=True)
    aff_ref[0] = aff

    val = val_ref[:, 0]
    oft = jnp.sum(val * aff[None, :, :], axis=2)
    oft_ref[0] = oft
    lgt_ref[0] = _bdot(clw_ref[...], oft) + clb_ref[...]

    bh = jnp.maximum(y3_ref[0] * bdp_ref[0:1, :] + bdp_ref[1:2, :], 0.0)
    bd = jnp.sum(bh * bdp_ref[2:3, :], axis=1) + bdp_ref[3, 0]
    bd_ref[0, 0] = bd


def _bn_affine(sums, g, be, count):
    mean = sums[0] / count
    var = sums[1] / count - mean * mean
    s = g / jnp.sqrt(var + EPS)
    return s, be - mean * s


@jax.jit
def kernel(xyz, feat, params):
    p = params
    f32 = jnp.float32
    xyz_pad = jnp.pad(xyz, ((0, 0), (0, 0), (0, 5)))
    xyzT = jnp.transpose(xyz_pad, (0, 2, 1))
    feat_pad = jnp.pad(feat.reshape(ROWS, G), ((0, 0), (0, 2)))

    w1t = jnp.pad(p['g_w1'].T, ((0, 2), (0, 0)))
    y1, sums1 = pl.pallas_call(
        _k1,
        grid=(NR,),
        in_specs=[_rows((RT, 8)), _full((8, C)), _full((1, C))],
        out_specs=[_rows((RT, C)), _full((2, C))],
        out_shape=[jax.ShapeDtypeStruct((ROWS, C), f32),
                   jax.ShapeDtypeStruct((2, C), f32)],
    )(feat_pad, w1t, p['g_b1'][None, :])

    s1, t1 = _bn_affine(sums1, p['g_g1'], p['g_be1'], ROWS)
    a1 = jnp.stack([s1, t1])

    y2, sums2 = pl.pallas_call(
        _k2,
        grid=(NR,),
        in_specs=[_rows((RT, C)), _full((2, C)), _full((C, C)), _full((1, C))],
        out_specs=[_rows((RT, C)), _full((2, C))],
        out_shape=[jax.ShapeDtypeStruct((ROWS, C), f32),
                   jax.ShapeDtypeStruct((2, C), f32)],
    )(y1, a1, p['g_w2'].T, p['g_b2'][None, :])

    s2, t2 = _bn_affine(sums2, p['g_g2'], p['g_be2'], ROWS)
    a2 = jnp.stack([s2, t2, p['sc_b'], p['sh_b'], p['q_b'], p['k_b'],
                    jnp.pad(p['bd_b1'], (0, 32))])

    scw = jnp.pad(p['sc_w'].T, ((0, 2), (0, 0)))
    shw = jnp.pad(p['sh_w'].T, ((0, 2), (0, 0)))
    geom, q, kf, qw2, y3, sums3 = pl.pallas_call(
        _k3,
        grid=(NR,),
        in_specs=[_rows((RT, C)), _rows((RT, 8)), _full((7, C)),
                  _full((8, C)), _full((8, C)), _full((C, C)), _full((C, C)),
                  _full((C, 32)), _full((C, C))],
        out_specs=[_rows((RT, C)), _rows((RT, C)), _rows((RT, C)),
                   _rows((RT, C)), _rows((RT, 32)), _full((2, 32))],
        out_shape=[jax.ShapeDtypeStruct((ROWS, C), f32),
                   jax.ShapeDtypeStruct((ROWS, C), f32),
                   jax.ShapeDtypeStruct((ROWS, C), f32),
                   jax.ShapeDtypeStruct((ROWS, C), f32),
                   jax.ShapeDtypeStruct((ROWS, 32), f32),
                   jax.ShapeDtypeStruct((2, 32), f32)],
    )(y2, feat_pad, a2, scw, shw, p['q_w'].T, p['k_w'].T,
      p['bd_w1'].T, p['rp_w2'])

    s3, t3 = _bn_affine(sums3, p['bd_g'], p['bd_be'], ROWS)

    qT_b = jnp.transpose(q.reshape(B, N, C), (0, 2, 1))
    qw2_b = qw2.reshape(B, N, C)
    y3_b = y3.reshape(B, N, 32)

    kidx, kidxg = pl.pallas_call(
        _k5,
        grid=(B, NT),
        in_specs=[
            pl.BlockSpec((1, T, 8), lambda b, i: (b, i, 0)),
            pl.BlockSpec((1, 8, N), lambda b, i: (b, 0, 0)),
        ],
        out_specs=[
            pl.BlockSpec((1, T, K), lambda b, i: (b, i, 0)),
            pl.BlockSpec((1, T, K), lambda b, i: (b, i, 0)),
        ],
        out_shape=[jax.ShapeDtypeStruct((B, N, K), jnp.int32),
                   jax.ShapeDtypeStruct((B, N, K), jnp.int32)],
    )(xyz_pad, xyzT)

    BNK = ROWS * K
    xcol = xyz.reshape(ROWS, 3)
    gat = pl.kernel(
        _sc_gather,
        mesh=plsc.VectorSubcoreMesh(core_axis_name="c", subcore_axis_name="s"),
        compiler_params=pltpu.CompilerParams(needs_layout_passes=False),
        out_type=[jax.ShapeDtypeStruct((BNK,), f32)] * 3,
        scratch_types=[
            pltpu.VMEM((ROWS,), f32),
            pltpu.VMEM((ROWS,), f32),
            pltpu.VMEM((ROWS,), f32),
            pltpu.VMEM((PW,), jnp.int32),
            pltpu.VMEM((PW,), f32),
            pltpu.VMEM((PW,), f32),
            pltpu.VMEM((PW,), f32),
        ],
    )
    g0, g1, g2 = gat(xcol[:, 0], xcol[:, 1], xcol[:, 2],
                     kidxg.reshape(BNK))

    xg, rps = pl.pallas_call(
        _k5b,
        grid=(B, NT),
        in_specs=[
            pl.BlockSpec((1, T, 8), lambda b, i: (b, i, 0)),
            pl.BlockSpec((1, T, K), lambda b, i: (b, i, 0)),
            pl.BlockSpec((1, T, K), lambda b, i: (b, i, 0)),
            pl.BlockSpec((1, T, K), lambda b, i: (b, i, 0)),
        ],
        out_specs=[
            pl.BlockSpec((1, 3, T, K), lambda b, i: (b, 0, i, 0)),
            pl.BlockSpec((16, 128), lambda b, i: (0, 0)),
        ],
        out_shape=[jax.ShapeDtypeStruct((B, 3, N, K), f32),
                   jax.ShapeDtypeStruct((16, 128), f32)],
    )(xyz_pad, g0.reshape(B, N, K), g1.reshape(B, N, K),
      g2.reshape(B, N, K))

    cnt = float(B * N * K)
    st = rps[:, 0]
    mu = st[0:3] / cnt
    m2 = jnp.array([[st[3], st[6], st[7]],
                    [st[6], st[4], st[8]],
                    [st[7], st[8], st[5]]]) / cnt
    cov = m2 - mu[:, None] * mu[None, :]
    w1 = p['rp_w1']
    mean_c = w1 @ mu + p['rp_b1']
    var_c = jnp.sum((w1 @ cov) * w1, axis=1)
    sc = p['rp_g'] / jnp.sqrt(var_c + EPS)
    dc = p['rp_b1'] * sc + p['rp_be'] - mean_c * sc
    pep = jnp.concatenate([w1.T * sc[None, :], dc[None, :],
                           jnp.zeros((4, C), f32)], axis=0)

    bdp = jnp.stack([s3, t3, p['bd_w2'][0],
                     jnp.broadcast_to(p['bd_b2'], (32,))])

    gat2 = pl.kernel(
        _sc_gather2,
        mesh=plsc.VectorSubcoreMesh(core_axis_name="c", subcore_axis_name="s"),
        compiler_params=pltpu.CompilerParams(needs_layout_passes=False),
        out_type=jax.ShapeDtypeStruct((C, BNK), f32),
        scratch_types=[
            pltpu.VMEM((ROWS,), f32),
            pltpu.VMEM((ROWS,), f32),
            pltpu.VMEM((CH,), jnp.int32),
            pltpu.VMEM((CH,), f32),
            pltpu.VMEM((CH,), f32),
        ],
    )
    idx_flat = kidxg.reshape(BNK)
    val = gat2(jnp.transpose(geom), idx_flat)
    val_b = val.reshape(C, B, N, K)
    kg = gat2(jnp.transpose(kf), idx_flat)
    kg_b = kg.reshape(C, B, N, K)

    aff, oft, lgt, bdy = pl.pallas_call(
        _k6,
        grid=(B, NT),
        in_specs=[
            pl.BlockSpec((1, T, 8), lambda b, i: (b, i, 0)),
            pl.BlockSpec((1, 3, T, K), lambda b, i: (b, 0, i, 0)),
            pl.BlockSpec((1, C, T), lambda b, i: (b, 0, i)),
            pl.BlockSpec((1, T, C), lambda b, i: (b, i, 0)),
            pl.BlockSpec((C, 1, T, K), lambda b, i: (0, b, i, 0)),
            pl.BlockSpec((C, 1, T, K), lambda b, i: (0, b, i, 0)),
            pl.BlockSpec((1, T, 32), lambda b, i: (b, i, 0)),
            _full((8, C)), _full((4, 32)), _full((NCLS, C)), _full((NCLS, 1)),
        ],
        out_specs=[
            pl.BlockSpec((1, T, K), lambda b, i: (b, i, 0)),
            pl.BlockSpec((1, C, T), lambda b, i: (b, 0, i)),
            pl.BlockSpec((1, NCLS, T), lambda b, i: (b, 0, i)),
            pl.BlockSpec((1, 1, T), lambda b, i: (b, 0, i)),
        ],
        out_shape=[jax.ShapeDtypeStruct((B, N, K), f32),
                   jax.ShapeDtypeStruct((B, C, N), f32),
                   jax.ShapeDtypeStruct((B, NCLS, N), f32),
                   jax.ShapeDtypeStruct((B, 1, N), f32)],
    )(xyz_pad, xg, qT_b, qw2_b, val_b, kg_b, y3_b,
      pep, bdp, p['cls_w'], p['cls_b'][:, None])

    out_flat = jnp.transpose(oft, (0, 2, 1)).reshape(ROWS, C)
    logits = jnp.transpose(lgt, (0, 2, 1)).reshape(ROWS, NCLS)
    return (logits, aff, kidx, out_flat, bdy)


# speedup vs baseline: 1.1263x; 1.1263x over previous
import functools

import jax
import jax.numpy as jnp
from jax import lax
from jax.experimental import pallas as pl
from jax.experimental.pallas import tpu as pltpu
from jax.experimental.pallas import tpu_sc as plsc

B, N = 4, 4096
C = 64
K = 16
G = 6
NCLS = 13
EPS = 1e-5
ROWS = B * N
RT = 2048
NR = ROWS // RT
T = 256
NT = N // T

_HI = jax.lax.Precision.HIGHEST


def _bdot(a, b):
    return jnp.dot(a.astype(jnp.bfloat16), b.astype(jnp.bfloat16),
                   preferred_element_type=jnp.float32)


def _full(shape):
    return pl.BlockSpec(shape, lambda *args: (0,) * len(shape))


def _rows(shape):
    return pl.BlockSpec(shape, lambda i: (i, 0))


def _k1(feat_ref, w_ref, b_ref, y1_ref, s_ref):
    y1 = _bdot(feat_ref[...], w_ref[...]) + b_ref[...]
    y1_ref[...] = y1

    @pl.when(pl.program_id(0) == 0)
    def _():
        s_ref[...] = jnp.zeros_like(s_ref)

    s_ref[0:1, :] += jnp.sum(y1, axis=0, keepdims=True)
    s_ref[1:2, :] += jnp.sum(y1 * y1, axis=0, keepdims=True)


def _k2(y1_ref, a1_ref, w_ref, b_ref, y2_ref, s_ref):
    h = jnp.maximum(y1_ref[...] * a1_ref[0:1, :] + a1_ref[1:2, :], 0.0)
    y2 = _bdot(h, w_ref[...]) + b_ref[...]
    y2_ref[...] = y2

    @pl.when(pl.program_id(0) == 0)
    def _():
        s_ref[...] = jnp.zeros_like(s_ref)

    s_ref[0:1, :] += jnp.sum(y2, axis=0, keepdims=True)
    s_ref[1:2, :] += jnp.sum(y2 * y2, axis=0, keepdims=True)


def _k3(y2_ref, feat_ref, a2_ref, scw_ref, shw_ref, qw_ref, kw_ref,
        bdw_ref, rp2_ref, geom_ref, q_ref, kf_ref, qw2_ref, y3_ref, s_ref):
    g0 = jnp.maximum(y2_ref[...] * a2_ref[0:1, :] + a2_ref[1:2, :], 0.0)
    f = feat_ref[...]
    scale = _bdot(f, scw_ref[...]) + a2_ref[2:3, :]
    shift = _bdot(f, shw_ref[...]) + a2_ref[3:4, :]
    geom = g0 * (scale + 1.0) + shift
    geom_ref[...] = geom
    q = _bdot(geom, qw_ref[...]) + a2_ref[4:5, :]
    q_ref[...] = q
    kf_ref[...] = _bdot(geom, kw_ref[...]) + a2_ref[5:6, :]
    qw2_ref[...] = jnp.dot(q, rp2_ref[...], precision=_HI)
    y3 = _bdot(geom, bdw_ref[...]) + a2_ref[6:7, 0:32]
    y3_ref[...] = y3

    @pl.when(pl.program_id(0) == 0)
    def _():
        s_ref[...] = jnp.zeros_like(s_ref)

    s_ref[0:1, :] += jnp.sum(y3, axis=0, keepdims=True)
    s_ref[1:2, :] += jnp.sum(y3 * y3, axis=0, keepdims=True)


def _k5(xt_ref, xall_ref, q_ref, kft_ref,
        kidx_ref, kidxg_ref, aq_ref):
    xt = xt_ref[0]
    xa = xall_ref[0]
    sq_t = (xt[:, 0:1] * xt[:, 0:1] + xt[:, 1:2] * xt[:, 1:2]) \
        + xt[:, 2:3] * xt[:, 2:3]
    sq_a = (xa[0:1, :] * xa[0:1, :] + xa[1:2, :] * xa[1:2, :]) \
        + xa[2:3, :] * xa[2:3, :]
    e = _bdot(xt, xa)
    d2 = (sq_t + sq_a) - 2.0 * e
    s_attn = _bdot(q_ref[0], kft_ref[0])

    lanef = jax.lax.broadcasted_iota(jnp.int32, (T, N), 1).astype(jnp.float32)
    lane16 = jax.lax.broadcasted_iota(jnp.int32, (T, K), 1)

    def body(k, carry):
        d2, kidxf, aq = carry
        mv = jnp.min(d2, axis=1, keepdims=True)
        cand = jnp.where(d2 == mv, lanef, jnp.inf)
        idxk = jnp.min(cand, axis=1, keepdims=True)
        onehot = cand == idxk
        aqk = jnp.sum(jnp.where(onehot, s_attn, 0.0), axis=1, keepdims=True)
        colk = lane16 == k
        kidxf = jnp.where(colk, idxk, kidxf)
        aq = jnp.where(colk, aqk, aq)
        d2 = jnp.where(onehot, jnp.inf, d2)
        return d2, kidxf, aq

    zf = jnp.zeros((T, K), jnp.float32)
    _, kidxf, aq = jax.lax.fori_loop(
        0, K, body,
        (d2, zf, zf))

    kidx = kidxf.astype(jnp.int32)
    kidx_ref[0] = kidx
    kidxg_ref[0] = kidx + pl.program_id(0) * N
    aq_ref[0] = aq


PW = (B * N * K) // 32


def _sc_gather(x0_hbm, x1_hbm, x2_hbm, idx_hbm, o0_hbm, o1_hbm, o2_hbm,
               x0_v, x1_v, x2_v, idx_v, o0_v, o1_v, o2_v):
    wid = lax.axis_index("s") * 2 + lax.axis_index("c")
    base = wid * PW
    pltpu.sync_copy(x0_hbm, x0_v)
    pltpu.sync_copy(x1_hbm, x1_v)
    pltpu.sync_copy(x2_hbm, x2_v)
    pltpu.sync_copy(idx_hbm.at[pl.ds(base, PW)], idx_v)

    def body(j, _):
        for u in range(4):
            s = pl.ds(j * 64 + u * 16, 16)
            iv = idx_v[s]
            o0_v[s] = plsc.load_gather(x0_v, [iv])
            o1_v[s] = plsc.load_gather(x1_v, [iv])
            o2_v[s] = plsc.load_gather(x2_v, [iv])
        return 0

    lax.fori_loop(0, PW // 64, body, 0)
    pltpu.sync_copy(o0_v, o0_hbm.at[pl.ds(base, PW)])
    pltpu.sync_copy(o1_v, o1_hbm.at[pl.ds(base, PW)])
    pltpu.sync_copy(o2_v, o2_hbm.at[pl.ds(base, PW)])


CH = 16384


def _sc_gather2(gco_hbm, idx_hbm, val_hbm, t0_v, t1_v, idx_v, o0_v, o1_v):
    wid = lax.axis_index("s") * 2 + lax.axis_index("c")
    c0 = wid * 2
    pltpu.sync_copy(gco_hbm.at[c0], t0_v)
    pltpu.sync_copy(gco_hbm.at[c0 + 1], t1_v)
    for cc in range(16):
        base = cc * CH
        pltpu.sync_copy(idx_hbm.at[pl.ds(base, CH)], idx_v)

        def gbody(j, _):
            for u in range(4):
                s = pl.ds(j * 64 + u * 16, 16)
                iv = idx_v[s]
                o0_v[s] = plsc.load_gather(t0_v, [iv])
                o1_v[s] = plsc.load_gather(t1_v, [iv])
            return 0

        lax.fori_loop(0, CH // 64, gbody, 0)
        pltpu.sync_copy(o0_v, val_hbm.at[c0, pl.ds(base, CH)])
        pltpu.sync_copy(o1_v, val_hbm.at[c0 + 1, pl.ds(base, CH)])


def _k5b(xt_ref, x0_ref, x1_ref, x2_ref, xg_ref, st_ref):
    xt = xt_ref[0]
    xg0 = x0_ref[0]
    xg1 = x1_ref[0]
    xg2 = x2_ref[0]
    xg_ref[0, 0] = xg0
    xg_ref[0, 1] = xg1
    xg_ref[0, 2] = xg2

    rx = xt[:, 0:1] - xg0
    ry = xt[:, 1:2] - xg1
    rz = xt[:, 2:3] - xg2

    @pl.when((pl.program_id(0) == 0) & (pl.program_id(1) == 0))
    def _():
        st_ref[...] = jnp.zeros_like(st_ref)

    def acc(j, v):
        st_ref[j:j + 1, :] += jnp.broadcast_to(jnp.sum(v), (1, 128))

    acc(0, rx)
    acc(1, ry)
    acc(2, rz)
    acc(3, rx * rx)
    acc(4, ry * ry)
    acc(5, rz * rz)
    acc(6, rx * ry)
    acc(7, rx * rz)
    acc(8, ry * rz)


def _k6(xt_ref, xg_ref, aq_ref, qw2_ref, val_ref, y3_ref,
        pep_ref, bdp_ref, clw_ref, clb_ref,
        aff_ref, oft_ref, lgt_ref, bd_ref):
    xt = xt_ref[0]
    rx = xt[:, 0:1] - xg_ref[0, 0]
    ry = xt[:, 1:2] - xg_ref[0, 1]
    rz = xt[:, 2:3] - xg_ref[0, 2]
    a0 = pep_ref[0:1, :].reshape(1, 1, C)
    a1 = pep_ref[1:2, :].reshape(1, 1, C)
    a2 = pep_ref[2:3, :].reshape(1, 1, C)
    dv = pep_ref[3:4, :].reshape(1, 1, C)
    pe = rx[:, :, None] * a0 + ry[:, :, None] * a1
    pe = pe + rz[:, :, None] * a2 + dv
    pe = jnp.maximum(pe, 0.0)
    qw2 = qw2_ref[0]
    posq = jnp.sum(pe * qw2[:, None, :], axis=2)
    attn = (aq_ref[0] + posq) * 0.125
    attn = attn - jnp.max(attn, axis=1, keepdims=True)
    ex = jnp.exp(attn)
    aff = ex / jnp.sum(ex, axis=1, keepdims=True)
    aff_ref[0] = aff

    val = val_ref[:, 0]
    oft = jnp.sum(val * aff[None, :, :], axis=2)
    oft_ref[0] = oft
    lgt_ref[0] = _bdot(clw_ref[...], oft) + clb_ref[...]

    bh = jnp.maximum(y3_ref[0] * bdp_ref[0:1, :] + bdp_ref[1:2, :], 0.0)
    bd = jnp.sum(bh * bdp_ref[2:3, :], axis=1) + bdp_ref[3, 0]
    bd_ref[0, 0] = bd


def _bn_affine(sums, g, be, count):
    mean = sums[0] / count
    var = sums[1] / count - mean * mean
    s = g / jnp.sqrt(var + EPS)
    return s, be - mean * s


@jax.jit
def kernel(xyz, feat, params):
    p = params
    f32 = jnp.float32
    xyz_pad = jnp.pad(xyz, ((0, 0), (0, 0), (0, 5)))
    xyzT = jnp.transpose(xyz_pad, (0, 2, 1))
    feat_pad = jnp.pad(feat.reshape(ROWS, G), ((0, 0), (0, 2)))

    w1t = jnp.pad(p['g_w1'].T, ((0, 2), (0, 0)))
    y1, sums1 = pl.pallas_call(
        _k1,
        grid=(NR,),
        in_specs=[_rows((RT, 8)), _full((8, C)), _full((1, C))],
        out_specs=[_rows((RT, C)), _full((2, C))],
        out_shape=[jax.ShapeDtypeStruct((ROWS, C), f32),
                   jax.ShapeDtypeStruct((2, C), f32)],
    )(feat_pad, w1t, p['g_b1'][None, :])

    s1, t1 = _bn_affine(sums1, p['g_g1'], p['g_be1'], ROWS)
    a1 = jnp.stack([s1, t1])

    y2, sums2 = pl.pallas_call(
        _k2,
        grid=(NR,),
        in_specs=[_rows((RT, C)), _full((2, C)), _full((C, C)), _full((1, C))],
        out_specs=[_rows((RT, C)), _full((2, C))],
        out_shape=[jax.ShapeDtypeStruct((ROWS, C), f32),
                   jax.ShapeDtypeStruct((2, C), f32)],
    )(y1, a1, p['g_w2'].T, p['g_b2'][None, :])

    s2, t2 = _bn_affine(sums2, p['g_g2'], p['g_be2'], ROWS)
    a2 = jnp.stack([s2, t2, p['sc_b'], p['sh_b'], p['q_b'], p['k_b'],
                    jnp.pad(p['bd_b1'], (0, 32))])

    scw = jnp.pad(p['sc_w'].T, ((0, 2), (0, 0)))
    shw = jnp.pad(p['sh_w'].T, ((0, 2), (0, 0)))
    geom, q, kf, qw2, y3, sums3 = pl.pallas_call(
        _k3,
        grid=(NR,),
        in_specs=[_rows((RT, C)), _rows((RT, 8)), _full((7, C)),
                  _full((8, C)), _full((8, C)), _full((C, C)), _full((C, C)),
                  _full((C, 32)), _full((C, C))],
        out_specs=[_rows((RT, C)), _rows((RT, C)), _rows((RT, C)),
                   _rows((RT, C)), _rows((RT, 32)), _full((2, 32))],
        out_shape=[jax.ShapeDtypeStruct((ROWS, C), f32),
                   jax.ShapeDtypeStruct((ROWS, C), f32),
                   jax.ShapeDtypeStruct((ROWS, C), f32),
                   jax.ShapeDtypeStruct((ROWS, C), f32),
                   jax.ShapeDtypeStruct((ROWS, 32), f32),
                   jax.ShapeDtypeStruct((2, 32), f32)],
    )(y2, feat_pad, a2, scw, shw, p['q_w'].T, p['k_w'].T,
      p['bd_w1'].T, p['rp_w2'])

    s3, t3 = _bn_affine(sums3, p['bd_g'], p['bd_be'], ROWS)

    q_b = q.reshape(B, N, C)
    kfT = jnp.transpose(kf.reshape(B, N, C), (0, 2, 1))
    geom_b = geom.reshape(B, N, C)
    qw2_b = qw2.reshape(B, N, C)
    y3_b = y3.reshape(B, N, 32)

    kidx, kidxg, aq = pl.pallas_call(
        _k5,
        grid=(B, NT),
        in_specs=[
            pl.BlockSpec((1, T, 8), lambda b, i: (b, i, 0)),
            pl.BlockSpec((1, 8, N), lambda b, i: (b, 0, 0)),
            pl.BlockSpec((1, T, C), lambda b, i: (b, i, 0)),
            pl.BlockSpec((1, C, N), lambda b, i: (b, 0, 0)),
        ],
        out_specs=[
            pl.BlockSpec((1, T, K), lambda b, i: (b, i, 0)),
            pl.BlockSpec((1, T, K), lambda b, i: (b, i, 0)),
            pl.BlockSpec((1, T, K), lambda b, i: (b, i, 0)),
        ],
        out_shape=[jax.ShapeDtypeStruct((B, N, K), jnp.int32),
                   jax.ShapeDtypeStruct((B, N, K), jnp.int32),
                   jax.ShapeDtypeStruct((B, N, K), f32)],
    )(xyz_pad, xyzT, q_b, kfT)

    BNK = ROWS * K
    xcol = xyz.reshape(ROWS, 3)
    gat = pl.kernel(
        _sc_gather,
        mesh=plsc.VectorSubcoreMesh(core_axis_name="c", subcore_axis_name="s"),
        compiler_params=pltpu.CompilerParams(needs_layout_passes=False),
        out_type=[jax.ShapeDtypeStruct((BNK,), f32)] * 3,
        scratch_types=[
            pltpu.VMEM((ROWS,), f32),
            pltpu.VMEM((ROWS,), f32),
            pltpu.VMEM((ROWS,), f32),
            pltpu.VMEM((PW,), jnp.int32),
            pltpu.VMEM((PW,), f32),
            pltpu.VMEM((PW,), f32),
            pltpu.VMEM((PW,), f32),
        ],
    )
    g0, g1, g2 = gat(xcol[:, 0], xcol[:, 1], xcol[:, 2],
                     kidxg.reshape(BNK))

    xg, rps = pl.pallas_call(
        _k5b,
        grid=(B, NT),
        in_specs=[
            pl.BlockSpec((1, T, 8), lambda b, i: (b, i, 0)),
            pl.BlockSpec((1, T, K), lambda b, i: (b, i, 0)),
            pl.BlockSpec((1, T, K), lambda b, i: (b, i, 0)),
            pl.BlockSpec((1, T, K), lambda b, i: (b, i, 0)),
        ],
        out_specs=[
            pl.BlockSpec((1, 3, T, K), lambda b, i: (b, 0, i, 0)),
            pl.BlockSpec((16, 128), lambda b, i: (0, 0)),
        ],
        out_shape=[jax.ShapeDtypeStruct((B, 3, N, K), f32),
                   jax.ShapeDtypeStruct((16, 128), f32)],
    )(xyz_pad, g0.reshape(B, N, K), g1.reshape(B, N, K),
      g2.reshape(B, N, K))

    cnt = float(B * N * K)
    st = rps[:, 0]
    mu = st[0:3] / cnt
    m2 = jnp.array([[st[3], st[6], st[7]],
                    [st[6], st[4], st[8]],
                    [st[7], st[8], st[5]]]) / cnt
    cov = m2 - mu[:, None] * mu[None, :]
    w1 = p['rp_w1']
    mean_c = w1 @ mu + p['rp_b1']
    var_c = jnp.sum((w1 @ cov) * w1, axis=1)
    sc = p['rp_g'] / jnp.sqrt(var_c + EPS)
    dc = p['rp_b1'] * sc + p['rp_be'] - mean_c * sc
    pep = jnp.concatenate([w1.T * sc[None, :], dc[None, :],
                           jnp.zeros((4, C), f32)], axis=0)

    bdp = jnp.stack([s3, t3, p['bd_w2'][0],
                     jnp.broadcast_to(p['bd_b2'], (32,))])

    val = pl.kernel(
        _sc_gather2,
        mesh=plsc.VectorSubcoreMesh(core_axis_name="c", subcore_axis_name="s"),
        compiler_params=pltpu.CompilerParams(needs_layout_passes=False),
        out_type=jax.ShapeDtypeStruct((C, BNK), f32),
        scratch_types=[
            pltpu.VMEM((ROWS,), f32),
            pltpu.VMEM((ROWS,), f32),
            pltpu.VMEM((CH,), jnp.int32),
            pltpu.VMEM((CH,), f32),
            pltpu.VMEM((CH,), f32),
        ],
    )(jnp.transpose(geom.reshape(ROWS, C)), kidxg.reshape(BNK))
    val_b = val.reshape(C, B, N, K)

    aff, oft, lgt, bdy = pl.pallas_call(
        _k6,
        grid=(B, NT),
        in_specs=[
            pl.BlockSpec((1, T, 8), lambda b, i: (b, i, 0)),
            pl.BlockSpec((1, 3, T, K), lambda b, i: (b, 0, i, 0)),
            pl.BlockSpec((1, T, K), lambda b, i: (b, i, 0)),
            pl.BlockSpec((1, T, C), lambda b, i: (b, i, 0)),
            pl.BlockSpec((C, 1, T, K), lambda b, i: (0, b, i, 0)),
            pl.BlockSpec((1, T, 32), lambda b, i: (b, i, 0)),
            _full((8, C)), _full((4, 32)), _full((NCLS, C)), _full((NCLS, 1)),
        ],
        out_specs=[
            pl.BlockSpec((1, T, K), lambda b, i: (b, i, 0)),
            pl.BlockSpec((1, C, T), lambda b, i: (b, 0, i)),
            pl.BlockSpec((1, NCLS, T), lambda b, i: (b, 0, i)),
            pl.BlockSpec((1, 1, T), lambda b, i: (b, 0, i)),
        ],
        out_shape=[jax.ShapeDtypeStruct((B, N, K), f32),
                   jax.ShapeDtypeStruct((B, C, N), f32),
                   jax.ShapeDtypeStruct((B, NCLS, N), f32),
                   jax.ShapeDtypeStruct((B, 1, N), f32)],
    )(xyz_pad, xg, aq, qw2_b, val_b, y3_b,
      pep, bdp, p['cls_w'], p['cls_b'][:, None])

    out_flat = jnp.transpose(oft, (0, 2, 1)).reshape(ROWS, C)
    logits = jnp.transpose(lgt, (0, 2, 1)).reshape(ROWS, NCLS)
    return (logits, aff, kidx, out_flat, bdy)
